# baseline (device time: 456365 ns/iter reference)
import jax
import jax.numpy as jnp
from jax import lax
from jax.experimental import pallas as pl
from jax.experimental.pallas import tpu as pltpu

M = 4096
K = 2048
N = 4096
NC = 16
Nc = N // NC
SLOTS = 4


def kernel(A, B):
    def body(a_hbm, b_ref, out_ref, a_vmem, send_buf, recv_buf,
             local_sem, send_sems, recv_sems, credit_sem):
        j = pl.program_id(0)
        my_x = lax.axis_index("x")
        my_y = lax.axis_index("y")
        neighbor = (my_x, 1 - my_y)

        def slot_rdma(s):
            return pltpu.make_async_remote_copy(
                src_ref=send_buf.at[s],
                dst_ref=recv_buf.at[s],
                send_sem=send_sems.at[s],
                recv_sem=recv_sems.at[s],
                device_id=neighbor,
                device_id_type=pl.DeviceIdType.MESH,
            )

        @pl.when(j == 0)
        def _():
            cp = pltpu.make_async_copy(a_hbm, a_vmem, local_sem)
            cp.start()
            cp.wait()
            barrier = pltpu.get_barrier_semaphore()
            pl.semaphore_signal(
                barrier, inc=1,
                device_id=neighbor, device_id_type=pl.DeviceIdType.MESH,
            )
            pl.semaphore_wait(barrier, 1)

        @pl.when(j < NC)
        def _():
            s = j % SLOTS

            @pl.when(j >= SLOTS)
            def _():
                slot_rdma(s).wait_send()
                pl.semaphore_wait(credit_sem, 1)

            send_buf[s] = jnp.dot(
                a_vmem[...], b_ref[...], preferred_element_type=jnp.float32
            ).astype(jnp.bfloat16)
            slot_rdma(s).start()

        @pl.when(j >= 2)
        def _():
            r = (j - 2) % SLOTS
            slot_rdma(r).wait_recv()
            out_ref[...] = (
                send_buf[r].astype(jnp.float32)
                + recv_buf[r].astype(jnp.float32)
            )
            pl.semaphore_signal(
                credit_sem, inc=1,
                device_id=neighbor, device_id_type=pl.DeviceIdType.MESH,
            )

        @pl.when(j == NC + 1)
        def _():
            for s in range(SLOTS):
                slot_rdma(s).wait_send()
            pl.semaphore_wait(credit_sem, SLOTS)

    return pl.pallas_call(
        body,
        grid=(NC + 2,),
        out_shape=jax.ShapeDtypeStruct((M, N), jnp.float32),
        in_specs=[
            pl.BlockSpec(memory_space=pl.ANY),
            pl.BlockSpec((K, Nc), lambda j: (0, jnp.minimum(j, NC - 1))),
        ],
        out_specs=pl.BlockSpec((M, Nc), lambda j: (0, jnp.maximum(j - 2, 0))),
        scratch_shapes=[
            pltpu.VMEM((M, K), jnp.bfloat16),
            pltpu.VMEM((SLOTS, M, Nc), jnp.bfloat16),
            pltpu.VMEM((SLOTS, M, Nc), jnp.bfloat16),
            pltpu.SemaphoreType.DMA,
            pltpu.SemaphoreType.DMA((SLOTS,)),
            pltpu.SemaphoreType.DMA((SLOTS,)),
            pltpu.SemaphoreType.REGULAR,
        ],
        compiler_params=pltpu.CompilerParams(
            dimension_semantics=("arbitrary",),
            collective_id=0,
            vmem_limit_bytes=100 * 1024 * 1024,
        ),
    )(A.astype(jnp.bfloat16), B.astype(jnp.bfloat16))


# device time: 378790 ns/iter; 1.2048x vs baseline; 1.2048x over previous
import jax
import jax.numpy as jnp
from jax import lax
from jax.experimental import pallas as pl
from jax.experimental.pallas import tpu as pltpu

M = 4096
K = 2048
N = 4096
NM = 16
MC = M // NM
SLOTS = 4
NB = 4
BC = (N // 2) // NB


def kernel(A, B):
    def body(a_hbm, b_hbm, out_ref, a_own, b_own, b_other, a_slots,
             cp_sems, bd_send, bd_recv, bf_send, bf_recv,
             a_send, a_recv, credit_sem):
        m = pl.program_id(0)
        my_x = lax.axis_index("x")
        my_y = lax.axis_index("y")
        y_nbr = (my_x, 1 - my_y)
        x_nbr = (1 - my_x, my_y)

        def a_rdma(q):
            return pltpu.make_async_remote_copy(
                src_ref=a_own.at[pl.ds(q * MC, MC), :],
                dst_ref=a_slots.at[q % SLOTS],
                send_sem=a_send.at[q % SLOTS],
                recv_sem=a_recv.at[q % SLOTS],
                device_id=y_nbr,
                device_id_type=pl.DeviceIdType.MESH,
            )

        def bd_rdma(c):
            return pltpu.make_async_remote_copy(
                src_ref=b_own.at[my_x, :, pl.ds(c * BC, BC)],
                dst_ref=b_other.at[my_x, :, pl.ds(c * BC, BC)],
                send_sem=bd_send.at[c],
                recv_sem=bd_recv.at[c],
                device_id=y_nbr,
                device_id_type=pl.DeviceIdType.MESH,
            )

        def bf_rdma(c):
            return pltpu.make_async_remote_copy(
                src_ref=b_other.at[my_x, :, pl.ds(c * BC, BC)],
                dst_ref=b_other.at[my_x, :, pl.ds(c * BC, BC)],
                send_sem=bf_send.at[c],
                recv_sem=bf_recv.at[c],
                device_id=x_nbr,
                device_id_type=pl.DeviceIdType.MESH,
            )

        @pl.when(m == 0)
        def _():
            cp_a = pltpu.make_async_copy(a_hbm, a_own, cp_sems.at[0])
            cp_b = pltpu.make_async_copy(b_hbm, b_own, cp_sems.at[1])
            cp_a.start()
            cp_b.start()
            cp_a.wait()
            cp_b.wait()
            barrier = pltpu.get_barrier_semaphore()
            for nbr in (y_nbr, x_nbr):
                pl.semaphore_signal(
                    barrier, inc=1,
                    device_id=nbr, device_id_type=pl.DeviceIdType.MESH,
                )
            pl.semaphore_wait(barrier, 2)

            for c in range(NB):
                bd_rdma(c).start()
            for q in range(SLOTS):
                a_rdma(q).start()
            for c in range(NB):
                bd_rdma(c).wait_recv()
                bf_rdma(c).start()
            for c in range(NB):
                bf_rdma(c).wait_recv()

        a_rdma(m).wait_recv()
        out_ref[:, pl.ds(0, N // 2)] = jnp.dot(
            a_own[pl.ds(m * MC, MC), :], b_own[0],
            preferred_element_type=jnp.float32,
        ) + jnp.dot(
            a_slots[m % SLOTS], b_other[0],
            preferred_element_type=jnp.float32,
        )
        out_ref[:, pl.ds(N // 2, N // 2)] = jnp.dot(
            a_own[pl.ds(m * MC, MC), :], b_own[1],
            preferred_element_type=jnp.float32,
        ) + jnp.dot(
            a_slots[m % SLOTS], b_other[1],
            preferred_element_type=jnp.float32,
        )
        pl.semaphore_signal(
            credit_sem, inc=1,
            device_id=y_nbr, device_id_type=pl.DeviceIdType.MESH,
        )

        @pl.when(m + SLOTS <= NM - 1)
        def _():
            pl.semaphore_wait(credit_sem, 1)
            a_rdma(m).wait_send()
            a_rdma(m + SLOTS).start()

        @pl.when(m == NM - 1)
        def _():
            for q in range(SLOTS):
                a_rdma(NM - SLOTS + q).wait_send()
            pl.semaphore_wait(credit_sem, SLOTS)
            for c in range(NB):
                bd_rdma(c).wait_send()
                bf_rdma(c).wait_send()

    return pl.pallas_call(
        body,
        grid=(NM,),
        out_shape=jax.ShapeDtypeStruct((M, N), jnp.float32),
        in_specs=[
            pl.BlockSpec(memory_space=pl.ANY),
            pl.BlockSpec(memory_space=pl.ANY),
        ],
        out_specs=pl.BlockSpec((MC, N), lambda m: (m, 0)),
        scratch_shapes=[
            pltpu.VMEM((M, K), jnp.bfloat16),
            pltpu.VMEM((2, K, N // 2), jnp.bfloat16),
            pltpu.VMEM((2, K, N // 2), jnp.bfloat16),
            pltpu.VMEM((SLOTS, MC, K), jnp.bfloat16),
            pltpu.SemaphoreType.DMA((2,)),
            pltpu.SemaphoreType.DMA((NB,)),
            pltpu.SemaphoreType.DMA((NB,)),
            pltpu.SemaphoreType.DMA((NB,)),
            pltpu.SemaphoreType.DMA((NB,)),
            pltpu.SemaphoreType.DMA((SLOTS,)),
            pltpu.SemaphoreType.DMA((SLOTS,)),
            pltpu.SemaphoreType.REGULAR,
        ],
        compiler_params=pltpu.CompilerParams(
            dimension_semantics=("arbitrary",),
            collective_id=0,
            vmem_limit_bytes=100 * 1024 * 1024,
        ),
    )(
        A.astype(jnp.bfloat16),
        jnp.stack(
            [B[:, : N // 2], B[:, N // 2 :]]
        ).astype(jnp.bfloat16),
    )


# device time: 369001 ns/iter; 1.2368x vs baseline; 1.0265x over previous
import jax
import jax.numpy as jnp
from jax import lax
from jax.experimental import pallas as pl
from jax.experimental.pallas import tpu as pltpu

M = 4096
K = 2048
N = 4096
NM = 16
MC = M // NM
SLOTS = 4
NB = 4
BC = (N // 2) // NB


def kernel(A, B):
    def body(a_hbm, b_hbm, out_ref, a_own, b_own, b_other, a_slots,
             cp_sems, bd_send, bd_recv, bf_send, bf_recv,
             a_send, a_recv, credit_sem):
        m = pl.program_id(0)
        my_x = lax.axis_index("x")
        my_y = lax.axis_index("y")
        y_nbr = (my_x, 1 - my_y)
        x_nbr = (1 - my_x, my_y)

        def a_rdma(q, from_hbm=False):
            src = a_hbm if from_hbm else a_own
            return pltpu.make_async_remote_copy(
                src_ref=src.at[pl.ds(q * MC, MC), :],
                dst_ref=a_slots.at[q % SLOTS],
                send_sem=a_send.at[q % SLOTS],
                recv_sem=a_recv.at[q % SLOTS],
                device_id=y_nbr,
                device_id_type=pl.DeviceIdType.MESH,
            )

        def bd_rdma(c):
            return pltpu.make_async_remote_copy(
                src_ref=b_hbm.at[my_x, :, pl.ds(c * BC, BC)],
                dst_ref=b_other.at[my_x, :, pl.ds(c * BC, BC)],
                send_sem=bd_send.at[c],
                recv_sem=bd_recv.at[c],
                device_id=y_nbr,
                device_id_type=pl.DeviceIdType.MESH,
            )

        def bf_rdma(c):
            return pltpu.make_async_remote_copy(
                src_ref=b_other.at[my_x, :, pl.ds(c * BC, BC)],
                dst_ref=b_other.at[my_x, :, pl.ds(c * BC, BC)],
                send_sem=bf_send.at[c],
                recv_sem=bf_recv.at[c],
                device_id=x_nbr,
                device_id_type=pl.DeviceIdType.MESH,
            )

        @pl.when(m == 0)
        def _():
            cp_a = pltpu.make_async_copy(a_hbm, a_own, cp_sems.at[0])
            cp_b = pltpu.make_async_copy(b_hbm, b_own, cp_sems.at[1])
            cp_a.start()
            cp_b.start()
            barrier = pltpu.get_barrier_semaphore()
            for nbr in (y_nbr, x_nbr):
                pl.semaphore_signal(
                    barrier, inc=1,
                    device_id=nbr, device_id_type=pl.DeviceIdType.MESH,
                )
            pl.semaphore_wait(barrier, 2)

            for c in range(NB):
                bd_rdma(c).start()
            for q in range(SLOTS):
                a_rdma(q, from_hbm=True).start()
            for c in range(NB):
                bd_rdma(c).wait_recv()
                bf_rdma(c).start()
            for c in range(NB):
                bf_rdma(c).wait_recv()
            cp_a.wait()
            cp_b.wait()

        a_rdma(m).wait_recv()
        out_ref[:, pl.ds(0, N // 2)] = jnp.dot(
            a_own[pl.ds(m * MC, MC), :], b_own[0],
            preferred_element_type=jnp.float32,
        ) + jnp.dot(
            a_slots[m % SLOTS], b_other[0],
            preferred_element_type=jnp.float32,
        )
        out_ref[:, pl.ds(N // 2, N // 2)] = jnp.dot(
            a_own[pl.ds(m * MC, MC), :], b_own[1],
            preferred_element_type=jnp.float32,
        ) + jnp.dot(
            a_slots[m % SLOTS], b_other[1],
            preferred_element_type=jnp.float32,
        )
        pl.semaphore_signal(
            credit_sem, inc=1,
            device_id=y_nbr, device_id_type=pl.DeviceIdType.MESH,
        )

        @pl.when(m + SLOTS <= NM - 1)
        def _():
            pl.semaphore_wait(credit_sem, 1)
            a_rdma(m).wait_send()
            a_rdma(m + SLOTS).start()

        @pl.when(m == NM - 1)
        def _():
            for q in range(SLOTS):
                a_rdma(NM - SLOTS + q).wait_send()
            pl.semaphore_wait(credit_sem, SLOTS)
            for c in range(NB):
                bd_rdma(c).wait_send()
                bf_rdma(c).wait_send()

    return pl.pallas_call(
        body,
        grid=(NM,),
        out_shape=jax.ShapeDtypeStruct((M, N), jnp.float32),
        in_specs=[
            pl.BlockSpec(memory_space=pl.ANY),
            pl.BlockSpec(memory_space=pl.ANY),
        ],
        out_specs=pl.BlockSpec((MC, N), lambda m: (m, 0)),
        scratch_shapes=[
            pltpu.VMEM((M, K), jnp.bfloat16),
            pltpu.VMEM((2, K, N // 2), jnp.bfloat16),
            pltpu.VMEM((2, K, N // 2), jnp.bfloat16),
            pltpu.VMEM((SLOTS, MC, K), jnp.bfloat16),
            pltpu.SemaphoreType.DMA((2,)),
            pltpu.SemaphoreType.DMA((NB,)),
            pltpu.SemaphoreType.DMA((NB,)),
            pltpu.SemaphoreType.DMA((NB,)),
            pltpu.SemaphoreType.DMA((NB,)),
            pltpu.SemaphoreType.DMA((SLOTS,)),
            pltpu.SemaphoreType.DMA((SLOTS,)),
            pltpu.SemaphoreType.REGULAR,
        ],
        compiler_params=pltpu.CompilerParams(
            dimension_semantics=("arbitrary",),
            collective_id=0,
            vmem_limit_bytes=100 * 1024 * 1024,
        ),
    )(
        A.astype(jnp.bfloat16),
        jnp.stack(
            [B[:, : N // 2], B[:, N // 2 :]]
        ).astype(jnp.bfloat16),
    )


# device time: 351224 ns/iter; 1.2994x vs baseline; 1.0506x over previous
import jax
import jax.numpy as jnp
from jax import lax
from jax.experimental import pallas as pl
from jax.experimental.pallas import tpu as pltpu

M = 4096
K = 2048
N = 4096
NM = 16
MC = M // NM
SLOTS = 4
NB = 4
BC = (N // 2) // NB


def kernel(A, B):
    def body(a_hbm, b_hbm, out_ref, a_own, b_own, b_other, a_slots, stg_a,
             cp_sems, bd_send, bd_recv, bf_send, bf_recv,
             a_send, a_recv, credit_sem):
        m = pl.program_id(0)
        my_x = lax.axis_index("x")
        my_y = lax.axis_index("y")
        y_nbr = (my_x, 1 - my_y)
        x_nbr = (1 - my_x, my_y)

        def a_rdma(q):
            return pltpu.make_async_remote_copy(
                src_ref=a_own.at[pl.ds(q * MC, MC), :],
                dst_ref=a_slots.at[q % SLOTS],
                send_sem=a_send.at[q % SLOTS],
                recv_sem=a_recv.at[q % SLOTS],
                device_id=y_nbr,
                device_id_type=pl.DeviceIdType.MESH,
            )

        def bd_rdma(c):
            return pltpu.make_async_remote_copy(
                src_ref=b_hbm.at[my_x, :, pl.ds(c * BC, BC)],
                dst_ref=b_other.at[my_x, :, pl.ds(c * BC, BC)],
                send_sem=bd_send.at[c],
                recv_sem=bd_recv.at[c],
                device_id=y_nbr,
                device_id_type=pl.DeviceIdType.MESH,
            )

        def bf_rdma(c):
            return pltpu.make_async_remote_copy(
                src_ref=b_other.at[my_x, :, pl.ds(c * BC, BC)],
                dst_ref=b_other.at[my_x, :, pl.ds(c * BC, BC)],
                send_sem=bf_send.at[c],
                recv_sem=bf_recv.at[c],
                device_id=x_nbr,
                device_id_type=pl.DeviceIdType.MESH,
            )

        def cast_a_chunk(q):
            cp = pltpu.make_async_copy(
                a_hbm.at[pl.ds(q * MC, MC), :], stg_a, cp_sems.at[0]
            )
            cp.start()
            cp.wait()
            a_own[pl.ds(q * MC, MC), :] = stg_a[...].astype(jnp.bfloat16)

        @pl.when(m == 0)
        def _():
            cp_b = pltpu.make_async_copy(b_hbm, b_own, cp_sems.at[1])
            cp_b.start()
            barrier = pltpu.get_barrier_semaphore()
            for nbr in (y_nbr, x_nbr):
                pl.semaphore_signal(
                    barrier, inc=1,
                    device_id=nbr, device_id_type=pl.DeviceIdType.MESH,
                )
            pl.semaphore_wait(barrier, 2)

            for c in range(NB):
                bd_rdma(c).start()
            for q in range(SLOTS):
                cast_a_chunk(q)
                a_rdma(q).start()
            for c in range(NB):
                bd_rdma(c).wait_recv()
                bf_rdma(c).start()
            for q in range(SLOTS, NM):
                cast_a_chunk(q)
            for c in range(NB):
                bf_rdma(c).wait_recv()
            cp_b.wait()

        a_rdma(m).wait_recv()
        out_ref[:, pl.ds(0, N // 2)] = jnp.dot(
            a_own[pl.ds(m * MC, MC), :], b_own[0],
            preferred_element_type=jnp.float32,
        ) + jnp.dot(
            a_slots[m % SLOTS], b_other[0],
            preferred_element_type=jnp.float32,
        )
        out_ref[:, pl.ds(N // 2, N // 2)] = jnp.dot(
            a_own[pl.ds(m * MC, MC), :], b_own[1],
            preferred_element_type=jnp.float32,
        ) + jnp.dot(
            a_slots[m % SLOTS], b_other[1],
            preferred_element_type=jnp.float32,
        )
        pl.semaphore_signal(
            credit_sem, inc=1,
            device_id=y_nbr, device_id_type=pl.DeviceIdType.MESH,
        )

        @pl.when(m + SLOTS <= NM - 1)
        def _():
            pl.semaphore_wait(credit_sem, 1)
            a_rdma(m).wait_send()
            a_rdma(m + SLOTS).start()

        @pl.when(m == NM - 1)
        def _():
            for q in range(SLOTS):
                a_rdma(NM - SLOTS + q).wait_send()
            pl.semaphore_wait(credit_sem, SLOTS)
            for c in range(NB):
                bd_rdma(c).wait_send()
                bf_rdma(c).wait_send()

    return pl.pallas_call(
        body,
        grid=(NM,),
        out_shape=jax.ShapeDtypeStruct((M, N), jnp.float32),
        in_specs=[
            pl.BlockSpec(memory_space=pl.ANY),
            pl.BlockSpec(memory_space=pl.ANY),
        ],
        out_specs=pl.BlockSpec((MC, N), lambda m: (m, 0)),
        scratch_shapes=[
            pltpu.VMEM((M, K), jnp.bfloat16),
            pltpu.VMEM((2, K, N // 2), jnp.bfloat16),
            pltpu.VMEM((2, K, N // 2), jnp.bfloat16),
            pltpu.VMEM((SLOTS, MC, K), jnp.bfloat16),
            pltpu.VMEM((MC, K), jnp.float32),
            pltpu.SemaphoreType.DMA((2,)),
            pltpu.SemaphoreType.DMA((NB,)),
            pltpu.SemaphoreType.DMA((NB,)),
            pltpu.SemaphoreType.DMA((NB,)),
            pltpu.SemaphoreType.DMA((NB,)),
            pltpu.SemaphoreType.DMA((SLOTS,)),
            pltpu.SemaphoreType.DMA((SLOTS,)),
            pltpu.SemaphoreType.REGULAR,
        ],
        compiler_params=pltpu.CompilerParams(
            dimension_semantics=("arbitrary",),
            collective_id=0,
            vmem_limit_bytes=100 * 1024 * 1024,
        ),
    )(
        A,
        jnp.stack(
            [B[:, : N // 2], B[:, N // 2 :]]
        ).astype(jnp.bfloat16),
    )
